# baseline (speedup 1.0000x reference)
import functools

import jax
import jax.numpy as jnp
from jax import lax
from jax.experimental import pallas as pl
from jax.experimental.pallas import tpu as pltpu
from jax.experimental.pallas import tpu_sc as plsc

_NUM_CORES = 2
_NUM_SUBCORES = 16
_NW = _NUM_CORES * _NUM_SUBCORES

_LANES = 128
_SUBS = 8
_TILE_WORDS = _SUBS * _LANES
_CHUNK_TILES = 128


@functools.partial(jax.jit, static_argnums=(2, 3))
def _gather2(x, table_t, B, D):
    b_per_w = B // _NW
    V = table_t.shape[1]
    n_tc = (V + _LANES - 1) // _LANES
    n_tr = D // _SUBS
    flat_len = n_tr * n_tc * _TILE_WORDS
    tr_stride = n_tc * _TILE_WORDS
    n_blocks = n_tr * n_tc
    per_w = (n_blocks + _NW - 1) // _NW
    _WIN = 16
    mesh = plsc.VectorSubcoreMesh(core_axis_name="c", subcore_axis_name="s")

    @functools.partial(
        pl.kernel,
        out_type=jax.ShapeDtypeStruct((n_blocks, _SUBS, _LANES), jnp.float32),
        mesh=mesh,
        scratch_types=[pltpu.SemaphoreType.DMA],
        compiler_params=pltpu.CompilerParams(disable_bounds_checks=True),
    )
    def ka(table_hbm, flat_hbm, sem):
        wid = lax.axis_index("s") * _NUM_CORES + lax.axis_index("c")
        lo = wid * per_w
        hi = jnp.minimum(lo + per_w, n_blocks)

        def start_one(t):
            tr = t // n_tc
            tc = t % n_tc
            pltpu.async_copy(
                table_hbm.at[
                    pl.ds(pl.multiple_of(tr * _SUBS, _SUBS), _SUBS),
                    pl.ds(pl.multiple_of(tc * _LANES, _LANES), _LANES),
                ],
                flat_hbm.at[t],
                sem,
            )

        def drain_one():
            pltpu.make_async_copy(
                table_hbm.at[pl.ds(0, _SUBS), pl.ds(0, _LANES)],
                flat_hbm.at[0],
                sem,
            ).wait()

        def body(t, carry):
            start_one(t)
            pl.when(t >= lo + _WIN)(drain_one)
            return carry

        lax.fori_loop(lo, hi, body, 0)

        def tail(i, carry):
            drain_one()
            return carry

        lax.fori_loop(0, jnp.minimum(_WIN, hi - lo), tail, 0)

    @functools.partial(
        pl.kernel,
        out_type=jax.ShapeDtypeStruct((D, B), jnp.float32),
        mesh=mesh,
        scratch_types=[
            pltpu.VMEM((b_per_w,), jnp.int32),
            pltpu.VMEM((D, b_per_w), jnp.int32),
            pltpu.VMEM((D, b_per_w), jnp.float32),
            pltpu.SemaphoreType.DMA,
        ],
        compiler_params=pltpu.CompilerParams(use_tc_tiling_on_sc=False),
    )
    def kb(x_hbm, flat_hbm, out_t_hbm, idx_v, offs_v, rows_v, sem):
        wid = lax.axis_index("s") * _NUM_CORES + lax.axis_index("c")
        base = wid * b_per_w
        pltpu.sync_copy(x_hbm.at[pl.ds(base, b_per_w)], idx_v)
        n16 = b_per_w // 16

        def obody(j, carry):
            v = idx_v[pl.ds(j * 16, 16)]
            b0 = (v >> 7) * _TILE_WORDS + (v & (_LANES - 1))
            for d in range(D):
                doff = (d // _SUBS) * tr_stride + (d % _SUBS) * _LANES
                offs_v[d, pl.ds(j * 16, 16)] = b0 + doff
            return carry

        lax.fori_loop(0, n16, obody, 0)
        copies = [
            pltpu.async_copy(flat_hbm.at[offs_v.at[d]], rows_v.at[d], sem)
            for d in range(D)
        ]
        for c in copies:
            c.wait()
        pltpu.sync_copy(rows_v, out_t_hbm.at[:, pl.ds(base, b_per_w)])

    flat = ka(table_t).reshape(flat_len)
    return kb(x, flat)


def kernel(x, table):
    (B,) = x.shape
    D = table.shape[1]
    out_t = _gather2(x.astype(jnp.int32), table.T, B, D)
    return out_t.T
